# baseline (speedup 1.0000x reference)
import functools

import jax
import jax.numpy as jnp
from jax import lax
from jax.experimental import pallas as pl
from jax.experimental.pallas import tpu as pltpu
from jax.experimental.pallas import tpu_sc as plsc

N_NODES = 100000
N_EDGES = 6400000
NUM_WORKERS = 32
CHUNK = 3200
NCHUNKS = N_EDGES // CHUNK
TPW = 64
VSTEPS = CHUNK // 16

_MAGIC = 0x5F3759DF


def _dr_from_packed(s_pack, d_pack):
    s_bf = plsc.bitcast(s_pack, jnp.bfloat16)
    d_bf = plsc.bitcast(d_pack, jnp.bfloat16)
    diff = s_bf - d_bf
    diff2 = diff * diff
    da2, db2 = plsc.unpack(diff2, format=plsc.PackFormat.INTERLEAVED)
    d2 = da2 + db2
    i = plsc.bitcast(d2, jnp.int32)
    y = plsc.bitcast(jnp.int32(_MAGIC) - (i >> 1), jnp.float32)
    y = y * (jnp.float32(1.5) - jnp.float32(0.5) * d2 * y * y)
    return d2 * y


def _make_kernel():
    mesh = plsc.VectorSubcoreMesh(core_axis_name="c", subcore_axis_name="s")

    @functools.partial(
        pl.kernel,
        out_type=jax.ShapeDtypeStruct((N_EDGES,), jnp.float32),
        mesh=mesh,
        compiler_params=pltpu.CompilerParams(needs_layout_passes=False),
        scratch_types=[
            pltpu.VMEM((N_NODES,), jnp.int32),
            pltpu.VMEM((2, CHUNK), jnp.int32),
            pltpu.VMEM((2, CHUNK), jnp.int32),
            pltpu.VMEM((CHUNK,), jnp.float32),
            pltpu.VMEM((CHUNK,), jnp.float32),
            pltpu.SemaphoreType.DMA,
            pltpu.SemaphoreType.DMA,
            pltpu.SemaphoreType.DMA,
            pltpu.SemaphoreType.DMA,
            pltpu.SemaphoreType.DMA,
        ],
    )
    def edge_dr(table_hbm, ei_hbm, out_hbm, tab_v, ids_v0, ids_v1,
                out_v0, out_v1, isem0, isem1, osem0, osem1, tsem):
        wid = lax.axis_index("s") * 2 + lax.axis_index("c")
        ids = (ids_v0, ids_v1)
        outs = (out_v0, out_v1)
        isems = (isem0, isem1)
        osems = (osem0, osem1)

        def chunk_id(t):
            return lax.rem(wid + t * NUM_WORKERS, NCHUNKS)

        def in_copy(k, b):
            return pltpu.make_async_copy(
                ei_hbm.at[:, pl.ds(k * CHUNK, CHUNK)], ids[b], isems[b])

        def out_copy(k, b):
            return pltpu.make_async_copy(
                outs[b], out_hbm.at[pl.ds(k * CHUNK, CHUNK)], osems[b])

        def compute(b):
            iv, ov = ids[b], outs[b]

            @plsc.parallel_loop(0, VSTEPS, unroll=16)
            def _(j):
                o = j * 16
                s_pack = plsc.load_gather(tab_v, [iv[0, pl.ds(o, 16)]])
                d_pack = plsc.load_gather(tab_v, [iv[1, pl.ds(o, 16)]])
                ov[pl.ds(o, 16)] = _dr_from_packed(s_pack, d_pack)

        tab_cp = pltpu.make_async_copy(table_hbm, tab_v, tsem)
        tab_cp.start()
        in_copy(chunk_id(0), 0).start()
        tab_cp.wait()

        def pair_body(p, carry):
            for b in range(2):
                t = 2 * p + b

                @pl.when(t + 1 < TPW)
                def _():
                    in_copy(chunk_id(t + 1), 1 - b).start()

                in_copy(chunk_id(t), b).wait()

                @pl.when(t >= 2)
                def _():
                    out_copy(chunk_id(t - 2), b).wait()

                compute(b)
                out_copy(chunk_id(t), b).start()
            return carry

        lax.fori_loop(0, TPW // 2, pair_body, 0)
        out_copy(chunk_id(TPW - 2), 0).wait()
        out_copy(chunk_id(TPW - 1), 1).wait()

    return edge_dr


_EDGE_DR = _make_kernel()


def kernel(center, edge_index):
    packed = lax.bitcast_convert_type(center.astype(jnp.bfloat16), jnp.int32)
    ei = edge_index.astype(jnp.int32)
    out = _EDGE_DR(packed, ei)
    return out.reshape(N_EDGES, 1)


# speedup vs baseline: 1.1238x; 1.1238x over previous
import functools

import jax
import jax.numpy as jnp
from jax import lax
from jax.experimental import pallas as pl
from jax.experimental.pallas import tpu as pltpu
from jax.experimental.pallas import tpu_sc as plsc

N_NODES = 100000
N_EDGES = 6400000
NUM_WORKERS = 32
CHUNK = 3200
NCHUNKS = N_EDGES // CHUNK
TPW = 64
VSTEPS = CHUNK // 16

_MAGIC = 0x5F3759DF


def _dr_from_packed(s_pack, d_pack):
    s_bf = plsc.bitcast(s_pack, jnp.bfloat16)
    d_bf = plsc.bitcast(d_pack, jnp.bfloat16)
    diff = s_bf - d_bf
    diff2 = diff * diff
    da2, db2 = plsc.unpack(diff2, format=plsc.PackFormat.INTERLEAVED)
    d2 = da2 + db2
    i = plsc.bitcast(d2, jnp.int32)
    y = plsc.bitcast(jnp.int32(_MAGIC) - (i >> 1), jnp.float32)
    y = y * (jnp.float32(1.5) - jnp.float32(0.5) * d2 * y * y)
    return d2 * y


def _make_kernel():
    mesh = plsc.VectorSubcoreMesh(core_axis_name="c", subcore_axis_name="s")

    @functools.partial(
        pl.kernel,
        out_type=jax.ShapeDtypeStruct((N_EDGES,), jnp.float32),
        mesh=mesh,
        compiler_params=pltpu.CompilerParams(
            needs_layout_passes=False, skip_device_barrier=True),
        scratch_types=[
            pltpu.VMEM((N_NODES,), jnp.int32),
            pltpu.VMEM((2, CHUNK), jnp.int32),
            pltpu.VMEM((2, CHUNK), jnp.int32),
            pltpu.VMEM((CHUNK,), jnp.float32),
            pltpu.VMEM((CHUNK,), jnp.float32),
            pltpu.SemaphoreType.DMA,
            pltpu.SemaphoreType.DMA,
            pltpu.SemaphoreType.DMA,
            pltpu.SemaphoreType.DMA,
            pltpu.SemaphoreType.DMA,
        ],
    )
    def edge_dr(table_hbm, ei_hbm, out_hbm, tab_v, ids_v0, ids_v1,
                out_v0, out_v1, isem0, isem1, osem0, osem1, tsem):
        wid = lax.axis_index("s") * 2 + lax.axis_index("c")
        ids = (ids_v0, ids_v1)
        outs = (out_v0, out_v1)
        isems = (isem0, isem1)
        osems = (osem0, osem1)

        def chunk_id(t):
            return lax.rem(wid + t * NUM_WORKERS, NCHUNKS)

        def in_copy(k, b):
            return pltpu.make_async_copy(
                ei_hbm.at[:, pl.ds(k * CHUNK, CHUNK)], ids[b], isems[b])

        def out_copy(k, b):
            return pltpu.make_async_copy(
                outs[b], out_hbm.at[pl.ds(k * CHUNK, CHUNK)], osems[b])

        def compute(b):
            iv, ov = ids[b], outs[b]

            @plsc.parallel_loop(0, VSTEPS, unroll=8)
            def _(j):
                o = j * 16
                s_pack = plsc.load_gather(tab_v, [iv[0, pl.ds(o, 16)]])
                d_pack = plsc.load_gather(tab_v, [iv[1, pl.ds(o, 16)]])
                ov[pl.ds(o, 16)] = _dr_from_packed(s_pack, d_pack)

        tab_cp = pltpu.make_async_copy(table_hbm, tab_v, tsem)
        tab_cp.start()
        in_copy(chunk_id(0), 0).start()
        tab_cp.wait()

        def pair_body(p, carry):
            for b in range(2):
                t = 2 * p + b

                @pl.when(t + 1 < TPW)
                def _():
                    in_copy(chunk_id(t + 1), 1 - b).start()

                in_copy(chunk_id(t), b).wait()

                @pl.when(t >= 2)
                def _():
                    out_copy(chunk_id(t - 2), b).wait()

                compute(b)
                out_copy(chunk_id(t), b).start()
            return carry

        lax.fori_loop(0, TPW // 2, pair_body, 0)
        out_copy(chunk_id(TPW - 2), 0).wait()
        out_copy(chunk_id(TPW - 1), 1).wait()

    return edge_dr


_EDGE_DR = _make_kernel()


def kernel(center, edge_index):
    packed = lax.bitcast_convert_type(center.astype(jnp.bfloat16), jnp.int32)
    ei = edge_index.astype(jnp.int32)
    out = _EDGE_DR(packed, ei)
    return out.reshape(N_EDGES, 1)


# speedup vs baseline: 1.1354x; 1.0103x over previous
import functools

import jax
import jax.numpy as jnp
from jax import lax
from jax.experimental import pallas as pl
from jax.experimental.pallas import tpu as pltpu
from jax.experimental.pallas import tpu_sc as plsc

N_NODES = 100000
N_EDGES = 6400000
NUM_WORKERS = 32
CHUNK = 3200
NCHUNKS = N_EDGES // CHUNK
TPW = 63
VSTEPS = CHUNK // 16

_MAGIC = 0x5F3759DF


def _dr_from_packed(s_pack, d_pack):
    s_bf = plsc.bitcast(s_pack, jnp.bfloat16)
    d_bf = plsc.bitcast(d_pack, jnp.bfloat16)
    diff = s_bf - d_bf
    diff2 = diff * diff
    da2, db2 = plsc.unpack(diff2, format=plsc.PackFormat.INTERLEAVED)
    d2 = da2 + db2
    i = plsc.bitcast(d2, jnp.int32)
    y = plsc.bitcast(jnp.int32(_MAGIC) - (i >> 1), jnp.float32)
    y = y * (jnp.float32(1.5) - jnp.float32(0.5) * d2 * y * y)
    return d2 * y


def _make_kernel():
    mesh = plsc.VectorSubcoreMesh(core_axis_name="c", subcore_axis_name="s")

    @functools.partial(
        pl.kernel,
        out_type=jax.ShapeDtypeStruct((N_EDGES,), jnp.float32),
        mesh=mesh,
        compiler_params=pltpu.CompilerParams(needs_layout_passes=False),
        scratch_types=[
            pltpu.VMEM((N_NODES,), jnp.int32),
            pltpu.VMEM((2, CHUNK), jnp.int32),
            pltpu.VMEM((2, CHUNK), jnp.int32),
            pltpu.VMEM((CHUNK,), jnp.float32),
            pltpu.VMEM((CHUNK,), jnp.float32),
            pltpu.SemaphoreType.DMA,
            pltpu.SemaphoreType.DMA,
            pltpu.SemaphoreType.DMA,
            pltpu.SemaphoreType.DMA,
            pltpu.SemaphoreType.DMA,
        ],
    )
    def edge_dr(table_hbm, ei_hbm, out_hbm, tab_v, ids_v0, ids_v1,
                out_v0, out_v1, isem0, isem1, osem0, osem1, tsem):
        wid = lax.axis_index("s") * 2 + lax.axis_index("c")
        ids = (ids_v0, ids_v1)
        outs = (out_v0, out_v1)
        isems = (isem0, isem1)
        osems = (osem0, osem1)

        def chunk_id(t):
            return wid + t * NUM_WORKERS

        has_last = wid < NCHUNKS - (TPW - 1) * NUM_WORKERS

        def in_copy(k, b):
            return pltpu.make_async_copy(
                ei_hbm.at[:, pl.ds(k * CHUNK, CHUNK)], ids[b], isems[b])

        def out_copy(k, b):
            return pltpu.make_async_copy(
                outs[b], out_hbm.at[pl.ds(k * CHUNK, CHUNK)], osems[b])

        def compute(b):
            iv, ov = ids[b], outs[b]

            @plsc.parallel_loop(0, VSTEPS, unroll=8)
            def _(j):
                o = j * 16
                s_pack = plsc.load_gather(tab_v, [iv[0, pl.ds(o, 16)]])
                d_pack = plsc.load_gather(tab_v, [iv[1, pl.ds(o, 16)]])
                ov[pl.ds(o, 16)] = _dr_from_packed(s_pack, d_pack)

        tab_cp = pltpu.make_async_copy(table_hbm, tab_v, tsem)
        tab_cp.start()
        in_copy(chunk_id(0), 0).start()
        tab_cp.wait()

        def pair_body(p, carry):
            for b in range(2):
                t = 2 * p + b

                @pl.when(jnp.logical_or(t + 1 < TPW - 1, has_last))
                def _():
                    in_copy(chunk_id(t + 1), 1 - b).start()

                in_copy(chunk_id(t), b).wait()

                @pl.when(t >= 2)
                def _():
                    out_copy(chunk_id(t - 2), b).wait()

                compute(b)
                out_copy(chunk_id(t), b).start()
            return carry

        lax.fori_loop(0, (TPW - 1) // 2, pair_body, 0)

        @pl.when(has_last)
        def _():
            in_copy(chunk_id(TPW - 1), 0).wait()
            out_copy(chunk_id(TPW - 3), 0).wait()
            compute(0)
            out_copy(chunk_id(TPW - 1), 0).start()

        @pl.when(jnp.logical_not(has_last))
        def _():
            out_copy(chunk_id(TPW - 3), 0).wait()

        out_copy(chunk_id(TPW - 2), 1).wait()

        @pl.when(has_last)
        def _():
            out_copy(chunk_id(TPW - 1), 0).wait()

    return edge_dr


_EDGE_DR = _make_kernel()


def kernel(center, edge_index):
    packed = lax.bitcast_convert_type(center.astype(jnp.bfloat16), jnp.int32)
    ei = edge_index.astype(jnp.int32)
    out = _EDGE_DR(packed, ei)
    return out.reshape(N_EDGES, 1)
